# baseline (device time: 50116 ns/iter reference)
import jax
import jax.numpy as jnp
from jax import lax
from jax.experimental import pallas as pl
from jax.experimental.pallas import tpu as pltpu

N_DEV = 8
B, Sq, Skv, Hq, Dh = 2, 128, 1024, 32, 64
HL = Hq // N_DEV
SKVL = Skv // N_DEV
DM = 512
WIN = 128
NSRC = 2
CH = (B * Sq) // N_DEV
BF = jnp.bfloat16
F32 = jnp.float32


def kernel(x, Wq, K_ext, V_ext, Wo):
    def body(x_ref, wq_ref, k_ref, v_ref, wo_ref, out_ref,
             kbuf, vbuf, kstage, vstage, kbf, vbf, pbuf, rs_buf, obuf,
             kv_send_sems, kv_recv_sems, loc_sems, stage_sems,
             rs_send_sems, rs_recv_sems, ag_send_sems, ag_recv_sems):
        me = lax.axis_index("i")

        bar = pltpu.get_barrier_semaphore()
        for o in range(1, N_DEV):
            pl.semaphore_signal(
                bar, inc=1,
                device_id=((me + o) % N_DEV,),
                device_id_type=pl.DeviceIdType.MESH,
            )

        @pl.when(me < NSRC)
        def _():
            pltpu.make_async_copy(k_ref, kstage, stage_sems.at[0]).start()
            pltpu.make_async_copy(v_ref, vstage, stage_sems.at[1]).start()
            pltpu.make_async_copy(k_ref, kstage, stage_sems.at[0]).wait()
            for j in range(N_DEV):
                kbf[j] = kstage[:, :, HL * j:HL * (j + 1), :].astype(BF)
            pltpu.make_async_copy(v_ref, vstage, stage_sems.at[1]).wait()
            for j in range(N_DEV):
                vbf[j] = vstage[:, :, HL * j:HL * (j + 1), :].astype(BF)

        xm = x_ref[...].reshape(B * Sq, DM)
        qm = lax.dot(xm.astype(BF), wq_ref[...].astype(BF),
                     preferred_element_type=F32)

        pl.semaphore_wait(bar, N_DEV - 1)

        def kv_rdma(src, j, jj, bf_ref, buf, t):
            return pltpu.make_async_remote_copy(
                src_ref=bf_ref.at[j],
                dst_ref=buf.at[src],
                send_sem=kv_send_sems.at[jj, t],
                recv_sem=kv_recv_sems.at[src, t],
                device_id=(j,),
                device_id_type=pl.DeviceIdType.MESH,
            )

        def loc_copy(src, bf_ref, buf, t):
            return pltpu.make_async_copy(
                bf_ref.at[src],
                buf.at[src],
                loc_sems.at[t],
            )

        for src in range(NSRC):
            @pl.when(me == src)
            def _(src=src):
                jj = 0
                for j in range(N_DEV):
                    if j == src:
                        continue
                    kv_rdma(src, j, jj, kbf, kbuf, 0).start()
                    kv_rdma(src, j, jj, vbf, vbuf, 1).start()
                    jj += 1
                loc_copy(src, kbf, kbuf, 0).start()
                loc_copy(src, vbf, vbuf, 1).start()

        for src in range(NSRC):
            @pl.when(me == src)
            def _(src=src):
                loc_copy(src, kbf, kbuf, 0).wait()
                loc_copy(src, vbf, vbuf, 1).wait()

            @pl.when(me != src)
            def _(src=src):
                for t, buf in ((0, kbuf), (1, vbuf)):
                    pltpu.make_async_remote_copy(
                        src_ref=buf.at[src],
                        dst_ref=buf.at[src],
                        send_sem=kv_send_sems.at[0, t],
                        recv_sem=kv_recv_sems.at[src, t],
                        device_id=(0,),
                        device_id_type=pl.DeviceIdType.MESH,
                    ).wait_recv()

        qi = lax.broadcasted_iota(jnp.int32, (Sq, NSRC * SKVL), 0)
        ki = lax.broadcasted_iota(jnp.int32, (Sq, NSRC * SKVL), 1)
        mask = ki <= qi + WIN

        ctx_rows = []
        for b in range(B):
            kb = jnp.concatenate([kbuf[0, b], kbuf[1, b]], axis=0)
            vb = jnp.concatenate([vbuf[0, b], vbuf[1, b]], axis=0)
            qb = qm[b * Sq:(b + 1) * Sq, :].reshape(Sq, HL, Dh)
            heads = []
            for h in range(HL):
                q = qb[:, h, :]
                k = kb[:, h, :]
                v = vb[:, h, :]
                s = lax.dot_general(
                    q.astype(BF), k,
                    (((1,), (1,)), ((), ())),
                    preferred_element_type=F32,
                ) * 0.125
                s = jnp.where(mask, s, -1e9)
                m = jnp.max(s, axis=1, keepdims=True)
                e = jnp.exp(s - m)
                w = e / jnp.sum(e, axis=1, keepdims=True)
                ctx = lax.dot(w.astype(BF), v,
                              preferred_element_type=F32)
                heads.append(ctx)
            ctx_rows.append(jnp.concatenate(heads, axis=1))
        ctxm = jnp.concatenate(ctx_rows, axis=0)

        partial = lax.dot(ctxm.astype(BF), wo_ref[...].astype(BF),
                          preferred_element_type=F32)
        pbuf[...] = partial.astype(BF)

        def rs_rdma(o):
            dest = (me + o) % N_DEV
            return pltpu.make_async_remote_copy(
                src_ref=pbuf.at[pl.ds(dest * CH, CH), :],
                dst_ref=rs_buf.at[o],
                send_sem=rs_send_sems.at[o - 1],
                recv_sem=rs_recv_sems.at[o],
                device_id=(dest,),
                device_id_type=pl.DeviceIdType.MESH,
            )

        def ag_rdma(o):
            return pltpu.make_async_remote_copy(
                src_ref=obuf.at[pl.ds(me * CH, CH), :],
                dst_ref=obuf.at[pl.ds(me * CH, CH), :],
                send_sem=ag_send_sems.at[o - 1],
                recv_sem=ag_recv_sems.at[o],
                device_id=((me + o) % N_DEV,),
                device_id_type=pl.DeviceIdType.MESH,
            )

        for o in range(1, N_DEV):
            rs_rdma(o).start()
        rs_buf[0] = pbuf[pl.ds(me * CH, CH), :]
        for o in range(1, N_DEV):
            rs_rdma(o).wait_recv()

        red = jnp.sum(rs_buf[...].astype(F32), axis=0)
        obuf[pl.ds(me * CH, CH), :] = red.astype(BF)

        for o in range(1, N_DEV):
            ag_rdma(o).start()
        for o in range(1, N_DEV):
            ag_rdma(o).wait_recv()

        out_ref[...] = obuf[...].astype(F32).reshape(B, Sq, DM)

        for o in range(1, N_DEV):
            rs_rdma(o).wait_send()
            ag_rdma(o).wait_send()
        for src in range(NSRC):
            @pl.when(me == src)
            def _(src=src):
                jj = 0
                for j in range(N_DEV):
                    if j == src:
                        continue
                    kv_rdma(src, j, jj, kbf, kbuf, 0).wait_send()
                    kv_rdma(src, j, jj, vbf, vbuf, 1).wait_send()
                    jj += 1

    return pl.pallas_call(
        body,
        out_shape=jax.ShapeDtypeStruct((B, Sq, DM), F32),
        in_specs=[
            pl.BlockSpec(memory_space=pltpu.VMEM),
            pl.BlockSpec(memory_space=pltpu.VMEM),
            pl.BlockSpec(memory_space=pl.ANY),
            pl.BlockSpec(memory_space=pl.ANY),
            pl.BlockSpec(memory_space=pltpu.VMEM),
        ],
        out_specs=pl.BlockSpec(memory_space=pltpu.VMEM),
        scratch_shapes=[
            pltpu.VMEM((NSRC, B, SKVL, HL, Dh), BF),
            pltpu.VMEM((NSRC, B, SKVL, HL, Dh), BF),
            pltpu.VMEM((B, SKVL, Hq, Dh), F32),
            pltpu.VMEM((B, SKVL, Hq, Dh), F32),
            pltpu.VMEM((N_DEV, B, SKVL, HL, Dh), BF),
            pltpu.VMEM((N_DEV, B, SKVL, HL, Dh), BF),
            pltpu.VMEM((B * Sq, DM), BF),
            pltpu.VMEM((N_DEV, CH, DM), BF),
            pltpu.VMEM((B * Sq, DM), BF),
            pltpu.SemaphoreType.DMA((N_DEV - 1, 2)),
            pltpu.SemaphoreType.DMA((NSRC, 2)),
            pltpu.SemaphoreType.DMA((2,)),
            pltpu.SemaphoreType.DMA((2,)),
            pltpu.SemaphoreType.DMA((N_DEV - 1,)),
            pltpu.SemaphoreType.DMA((N_DEV,)),
            pltpu.SemaphoreType.DMA((N_DEV - 1,)),
            pltpu.SemaphoreType.DMA((N_DEV,)),
        ],
        compiler_params=pltpu.CompilerParams(collective_id=0),
    )(x, Wq, K_ext, V_ext, Wo)


# device time: 48943 ns/iter; 1.0240x vs baseline; 1.0240x over previous
import jax
import jax.numpy as jnp
from jax import lax
from jax.experimental import pallas as pl
from jax.experimental.pallas import tpu as pltpu

N_DEV = 8
B, Sq, Skv, Hq, Dh = 2, 128, 1024, 32, 64
HL = Hq // N_DEV
SKVL = Skv // N_DEV
DM = 512
WIN = 128
NSRC = 2
CH = (B * Sq) // N_DEV
BF = jnp.bfloat16
F32 = jnp.float32


def kernel(x, Wq, K_ext, V_ext, Wo):
    def body(x_ref, wq_ref, k_ref, v_ref, wo_ref, out_ref,
             kbuf, vbuf, kstage, vstage, kbf, vbf, pbuf, rs_buf, obuf,
             kv_send_sems, kv_recv_sems, loc_sems, stage_sems,
             rs_send_sems, rs_recv_sems, ag_send_sems, ag_recv_sems):
        me = lax.axis_index("i")

        bar = pltpu.get_barrier_semaphore()
        for o in range(1, N_DEV):
            pl.semaphore_signal(
                bar, inc=1,
                device_id=((me + o) % N_DEV,),
                device_id_type=pl.DeviceIdType.MESH,
            )

        @pl.when(me < NSRC)
        def _():
            pltpu.make_async_copy(k_ref, kstage, stage_sems.at[0]).start()
            pltpu.make_async_copy(v_ref, vstage, stage_sems.at[1]).start()
            pltpu.make_async_copy(k_ref, kstage, stage_sems.at[0]).wait()
            for j in range(N_DEV):
                kbf[j] = kstage[:, :, HL * j:HL * (j + 1), :].astype(BF)
            pltpu.make_async_copy(v_ref, vstage, stage_sems.at[1]).wait()
            for j in range(N_DEV):
                vbf[j] = vstage[:, :, HL * j:HL * (j + 1), :].astype(BF)

        xm = x_ref[...].reshape(B * Sq, DM)
        qm = lax.dot(xm.astype(BF), wq_ref[...].astype(BF),
                     preferred_element_type=F32)

        pl.semaphore_wait(bar, N_DEV - 1)

        def kv_rdma(src, j, jj, bf_ref, buf, t):
            return pltpu.make_async_remote_copy(
                src_ref=bf_ref.at[j],
                dst_ref=buf.at[src],
                send_sem=kv_send_sems.at[jj, t],
                recv_sem=kv_recv_sems.at[src, t],
                device_id=(j,),
                device_id_type=pl.DeviceIdType.MESH,
            )

        def loc_copy(src, bf_ref, buf, t):
            return pltpu.make_async_copy(
                bf_ref.at[src],
                buf.at[src],
                loc_sems.at[t],
            )

        for src in range(NSRC):
            @pl.when(me == src)
            def _(src=src):
                jj = 0
                for j in range(N_DEV):
                    if j == src:
                        continue
                    kv_rdma(src, j, jj, kbf, kbuf, 0).start()
                    jj += 1
                loc_copy(src, kbf, kbuf, 0).start()
                jj = 0
                for j in range(N_DEV):
                    if j == src:
                        continue
                    kv_rdma(src, j, jj, vbf, vbuf, 1).start()
                    jj += 1
                loc_copy(src, vbf, vbuf, 1).start()

        def wait_kv(t, buf):
            for src in range(NSRC):
                @pl.when(me == src)
                def _(src=src):
                    loc_copy(src, kbf if t == 0 else vbf, buf, t).wait()

                @pl.when(me != src)
                def _(src=src):
                    pltpu.make_async_remote_copy(
                        src_ref=buf.at[src],
                        dst_ref=buf.at[src],
                        send_sem=kv_send_sems.at[0, t],
                        recv_sem=kv_recv_sems.at[src, t],
                        device_id=(0,),
                        device_id_type=pl.DeviceIdType.MESH,
                    ).wait_recv()

        qi = lax.broadcasted_iota(jnp.int32, (Sq, NSRC * SKVL), 0)
        ki = lax.broadcasted_iota(jnp.int32, (Sq, NSRC * SKVL), 1)
        mask = ki <= qi + WIN

        wait_kv(0, kbuf)
        weights = []
        for b in range(B):
            kb = jnp.concatenate([kbuf[0, b], kbuf[1, b]], axis=0)
            qb = qm[b * Sq:(b + 1) * Sq, :].reshape(Sq, HL, Dh)
            for h in range(HL):
                q = qb[:, h, :]
                k = kb[:, h, :]
                s = lax.dot_general(
                    q.astype(BF), k,
                    (((1,), (1,)), ((), ())),
                    preferred_element_type=F32,
                ) * 0.125
                s = jnp.where(mask, s, -1e9)
                m = jnp.max(s, axis=1, keepdims=True)
                e = jnp.exp(s - m)
                w = e / jnp.sum(e, axis=1, keepdims=True)
                weights.append(w.astype(BF))

        wait_kv(1, vbuf)
        ctx_rows = []
        for b in range(B):
            vb = jnp.concatenate([vbuf[0, b], vbuf[1, b]], axis=0)
            heads = []
            for h in range(HL):
                ctx = lax.dot(weights[b * HL + h], vb[:, h, :],
                              preferred_element_type=F32)
                heads.append(ctx)
            ctx_rows.append(jnp.concatenate(heads, axis=1))
        ctxm = jnp.concatenate(ctx_rows, axis=0)

        partial = lax.dot(ctxm.astype(BF), wo_ref[...].astype(BF),
                          preferred_element_type=F32)
        pbuf[...] = partial.astype(BF)

        def rs_rdma(o):
            dest = (me + o) % N_DEV
            return pltpu.make_async_remote_copy(
                src_ref=pbuf.at[pl.ds(dest * CH, CH), :],
                dst_ref=rs_buf.at[o],
                send_sem=rs_send_sems.at[o - 1],
                recv_sem=rs_recv_sems.at[o],
                device_id=(dest,),
                device_id_type=pl.DeviceIdType.MESH,
            )

        def ag_rdma(o):
            return pltpu.make_async_remote_copy(
                src_ref=obuf.at[pl.ds(me * CH, CH), :],
                dst_ref=obuf.at[pl.ds(me * CH, CH), :],
                send_sem=ag_send_sems.at[o - 1],
                recv_sem=ag_recv_sems.at[o],
                device_id=((me + o) % N_DEV,),
                device_id_type=pl.DeviceIdType.MESH,
            )

        for o in range(1, N_DEV):
            rs_rdma(o).start()
        rs_buf[0] = pbuf[pl.ds(me * CH, CH), :]
        for o in range(1, N_DEV):
            rs_rdma(o).wait_recv()

        red = jnp.sum(rs_buf[...].astype(F32), axis=0)
        obuf[pl.ds(me * CH, CH), :] = red.astype(BF)

        for o in range(1, N_DEV):
            ag_rdma(o).start()
        for o in range(1, N_DEV):
            ag_rdma(o).wait_recv()

        out_ref[...] = obuf[...].astype(F32).reshape(B, Sq, DM)

        for o in range(1, N_DEV):
            rs_rdma(o).wait_send()
            ag_rdma(o).wait_send()
        for src in range(NSRC):
            @pl.when(me == src)
            def _(src=src):
                jj = 0
                for j in range(N_DEV):
                    if j == src:
                        continue
                    kv_rdma(src, j, jj, kbf, kbuf, 0).wait_send()
                    kv_rdma(src, j, jj, vbf, vbuf, 1).wait_send()
                    jj += 1

    return pl.pallas_call(
        body,
        out_shape=jax.ShapeDtypeStruct((B, Sq, DM), F32),
        in_specs=[
            pl.BlockSpec(memory_space=pltpu.VMEM),
            pl.BlockSpec(memory_space=pltpu.VMEM),
            pl.BlockSpec(memory_space=pl.ANY),
            pl.BlockSpec(memory_space=pl.ANY),
            pl.BlockSpec(memory_space=pltpu.VMEM),
        ],
        out_specs=pl.BlockSpec(memory_space=pltpu.VMEM),
        scratch_shapes=[
            pltpu.VMEM((NSRC, B, SKVL, HL, Dh), BF),
            pltpu.VMEM((NSRC, B, SKVL, HL, Dh), BF),
            pltpu.VMEM((B, SKVL, Hq, Dh), F32),
            pltpu.VMEM((B, SKVL, Hq, Dh), F32),
            pltpu.VMEM((N_DEV, B, SKVL, HL, Dh), BF),
            pltpu.VMEM((N_DEV, B, SKVL, HL, Dh), BF),
            pltpu.VMEM((B * Sq, DM), BF),
            pltpu.VMEM((N_DEV, CH, DM), BF),
            pltpu.VMEM((B * Sq, DM), BF),
            pltpu.SemaphoreType.DMA((N_DEV - 1, 2)),
            pltpu.SemaphoreType.DMA((NSRC, 2)),
            pltpu.SemaphoreType.DMA((2,)),
            pltpu.SemaphoreType.DMA((2,)),
            pltpu.SemaphoreType.DMA((N_DEV - 1,)),
            pltpu.SemaphoreType.DMA((N_DEV,)),
            pltpu.SemaphoreType.DMA((N_DEV - 1,)),
            pltpu.SemaphoreType.DMA((N_DEV,)),
        ],
        compiler_params=pltpu.CompilerParams(collective_id=0),
    )(x, Wq, K_ext, V_ext, Wo)


# device time: 44348 ns/iter; 1.1301x vs baseline; 1.1036x over previous
import jax
import jax.numpy as jnp
from jax import lax
from jax.experimental import pallas as pl
from jax.experimental.pallas import tpu as pltpu

N_DEV = 8
B, Sq, Skv, Hq, Dh = 2, 128, 1024, 32, 64
HL = Hq // N_DEV
SKVL = Skv // N_DEV
DM = 512
WIN = 128
NSRC = 2
CH = (B * Sq) // N_DEV
BF = jnp.bfloat16
F32 = jnp.float32


def kernel(x, Wq, K_ext, V_ext, Wo):
    Kb = K_ext.reshape(B, SKVL, N_DEV, HL, Dh).transpose(2, 0, 1, 3, 4).astype(BF)
    Vb = V_ext.reshape(B, SKVL, N_DEV, HL, Dh).transpose(2, 0, 1, 3, 4).astype(BF)

    def body(x_ref, wq_ref, kb_ref, vb_ref, wo_ref, out_ref,
             kbuf, vbuf, pbuf, rs_buf, obuf,
             kv_send_sems, kv_recv_sems, loc_sems,
             rs_send_sems, rs_recv_sems, ag_send_sems, ag_recv_sems):
        me = lax.axis_index("i")

        bar = pltpu.get_barrier_semaphore()
        for o in range(1, N_DEV):
            pl.semaphore_signal(
                bar, inc=1,
                device_id=((me + o) % N_DEV,),
                device_id_type=pl.DeviceIdType.MESH,
            )

        xm = x_ref[...].reshape(B * Sq, DM)
        qm = lax.dot(xm.astype(BF), wq_ref[...].astype(BF),
                     preferred_element_type=F32)

        pl.semaphore_wait(bar, N_DEV - 1)

        def kv_rdma(src, j, jj, ext_ref, buf, t):
            return pltpu.make_async_remote_copy(
                src_ref=ext_ref.at[j],
                dst_ref=buf.at[src],
                send_sem=kv_send_sems.at[jj, t],
                recv_sem=kv_recv_sems.at[src, t],
                device_id=(j,),
                device_id_type=pl.DeviceIdType.MESH,
            )

        def loc_copy(src, ext_ref, buf, t):
            return pltpu.make_async_copy(
                ext_ref.at[src],
                buf.at[src],
                loc_sems.at[t],
            )

        for src in range(NSRC):
            @pl.when(me == src)
            def _(src=src):
                jj = 0
                for j in range(N_DEV):
                    if j == src:
                        continue
                    kv_rdma(src, j, jj, kb_ref, kbuf, 0).start()
                    jj += 1
                loc_copy(src, kb_ref, kbuf, 0).start()
                jj = 0
                for j in range(N_DEV):
                    if j == src:
                        continue
                    kv_rdma(src, j, jj, vb_ref, vbuf, 1).start()
                    jj += 1
                loc_copy(src, vb_ref, vbuf, 1).start()

        def wait_kv(t, buf):
            for src in range(NSRC):
                @pl.when(me == src)
                def _(src=src):
                    loc_copy(src, kb_ref if t == 0 else vb_ref, buf, t).wait()

                @pl.when(me != src)
                def _(src=src):
                    pltpu.make_async_remote_copy(
                        src_ref=buf.at[src],
                        dst_ref=buf.at[src],
                        send_sem=kv_send_sems.at[0, t],
                        recv_sem=kv_recv_sems.at[src, t],
                        device_id=(0,),
                        device_id_type=pl.DeviceIdType.MESH,
                    ).wait_recv()

        qi = lax.broadcasted_iota(jnp.int32, (Sq, NSRC * SKVL), 0)
        ki = lax.broadcasted_iota(jnp.int32, (Sq, NSRC * SKVL), 1)
        mask = ki <= qi + WIN

        wait_kv(0, kbuf)
        weights = []
        for b in range(B):
            kb = jnp.concatenate([kbuf[0, b], kbuf[1, b]], axis=0)
            qb = qm[b * Sq:(b + 1) * Sq, :].reshape(Sq, HL, Dh)
            for h in range(HL):
                q = qb[:, h, :]
                k = kb[:, h, :]
                s = lax.dot_general(
                    q.astype(BF), k,
                    (((1,), (1,)), ((), ())),
                    preferred_element_type=F32,
                ) * 0.125
                s = jnp.where(mask, s, -1e9)
                m = jnp.max(s, axis=1, keepdims=True)
                e = jnp.exp(s - m)
                w = e / jnp.sum(e, axis=1, keepdims=True)
                weights.append(w.astype(BF))

        wait_kv(1, vbuf)
        ctx_rows = []
        for b in range(B):
            vb = jnp.concatenate([vbuf[0, b], vbuf[1, b]], axis=0)
            heads = []
            for h in range(HL):
                ctx = lax.dot(weights[b * HL + h], vb[:, h, :],
                              preferred_element_type=F32)
                heads.append(ctx)
            ctx_rows.append(jnp.concatenate(heads, axis=1))
        ctxm = jnp.concatenate(ctx_rows, axis=0)

        partial = lax.dot(ctxm.astype(BF), wo_ref[...].astype(BF),
                          preferred_element_type=F32)
        pbuf[...] = partial.astype(BF)

        def rs_rdma(o):
            dest = (me + o) % N_DEV
            return pltpu.make_async_remote_copy(
                src_ref=pbuf.at[pl.ds(dest * CH, CH), :],
                dst_ref=rs_buf.at[o],
                send_sem=rs_send_sems.at[o - 1],
                recv_sem=rs_recv_sems.at[o],
                device_id=(dest,),
                device_id_type=pl.DeviceIdType.MESH,
            )

        def ag_rdma(o):
            return pltpu.make_async_remote_copy(
                src_ref=obuf.at[pl.ds(me * CH, CH), :],
                dst_ref=obuf.at[pl.ds(me * CH, CH), :],
                send_sem=ag_send_sems.at[o - 1],
                recv_sem=ag_recv_sems.at[o],
                device_id=((me + o) % N_DEV,),
                device_id_type=pl.DeviceIdType.MESH,
            )

        for o in range(1, N_DEV):
            rs_rdma(o).start()
        rs_buf[0] = pbuf[pl.ds(me * CH, CH), :]
        for o in range(1, N_DEV):
            rs_rdma(o).wait_recv()

        red = jnp.sum(rs_buf[...].astype(F32), axis=0)
        obuf[pl.ds(me * CH, CH), :] = red.astype(BF)

        for o in range(1, N_DEV):
            ag_rdma(o).start()
        for o in range(1, N_DEV):
            ag_rdma(o).wait_recv()

        out_ref[...] = obuf[...].astype(F32).reshape(B, Sq, DM)

        for o in range(1, N_DEV):
            rs_rdma(o).wait_send()
            ag_rdma(o).wait_send()
        for src in range(NSRC):
            @pl.when(me == src)
            def _(src=src):
                jj = 0
                for j in range(N_DEV):
                    if j == src:
                        continue
                    kv_rdma(src, j, jj, kb_ref, kbuf, 0).wait_send()
                    kv_rdma(src, j, jj, vb_ref, vbuf, 1).wait_send()
                    jj += 1

    return pl.pallas_call(
        body,
        out_shape=jax.ShapeDtypeStruct((B, Sq, DM), F32),
        in_specs=[
            pl.BlockSpec(memory_space=pltpu.VMEM),
            pl.BlockSpec(memory_space=pltpu.VMEM),
            pl.BlockSpec(memory_space=pl.ANY),
            pl.BlockSpec(memory_space=pl.ANY),
            pl.BlockSpec(memory_space=pltpu.VMEM),
        ],
        out_specs=pl.BlockSpec(memory_space=pltpu.VMEM),
        scratch_shapes=[
            pltpu.VMEM((NSRC, B, SKVL, HL, Dh), BF),
            pltpu.VMEM((NSRC, B, SKVL, HL, Dh), BF),
            pltpu.VMEM((B * Sq, DM), BF),
            pltpu.VMEM((N_DEV, CH, DM), BF),
            pltpu.VMEM((B * Sq, DM), BF),
            pltpu.SemaphoreType.DMA((N_DEV - 1, 2)),
            pltpu.SemaphoreType.DMA((NSRC, 2)),
            pltpu.SemaphoreType.DMA((2,)),
            pltpu.SemaphoreType.DMA((N_DEV - 1,)),
            pltpu.SemaphoreType.DMA((N_DEV,)),
            pltpu.SemaphoreType.DMA((N_DEV - 1,)),
            pltpu.SemaphoreType.DMA((N_DEV,)),
        ],
        compiler_params=pltpu.CompilerParams(collective_id=0),
    )(x, Wq, Kb, Vb, Wo)


# device time: 28062 ns/iter; 1.7859x vs baseline; 1.5804x over previous
import jax
import jax.numpy as jnp
from jax import lax
from jax.experimental import pallas as pl
from jax.experimental.pallas import tpu as pltpu

N_DEV = 8
B, Sq, Skv, Hq, Dh = 2, 128, 1024, 32, 64
HL = Hq // N_DEV
SKVL = Skv // N_DEV
DM = 512
WIN = 128
NSRC = 2
CH = (B * Sq) // N_DEV
BF = jnp.bfloat16
F32 = jnp.float32


def kernel(x, Wq, K_ext, V_ext, Wo):
    Kb = jnp.transpose(K_ext, (0, 2, 3, 1)).astype(BF)
    Vb = jnp.transpose(V_ext, (0, 2, 3, 1)).astype(BF)

    def body(x_ref, wq_ref, kb_ref, vb_ref, wo_ref, out_ref,
             kbuf, vbuf, pbuf, rs_buf, obuf,
             kv_send_sems, kv_recv_sems, loc_sems,
             rs_send_sems, rs_recv_sems, ag_send_sems, ag_recv_sems):
        me = lax.axis_index("i")

        bar = pltpu.get_barrier_semaphore()
        for o in range(1, N_DEV):
            pl.semaphore_signal(
                bar, inc=1,
                device_id=((me + o) % N_DEV,),
                device_id_type=pl.DeviceIdType.MESH,
            )

        xm = x_ref[...].reshape(B * Sq, DM)
        qm = lax.dot(xm.astype(BF), wq_ref[...].astype(BF),
                     preferred_element_type=F32)

        pl.semaphore_wait(bar, N_DEV - 1)

        def kv_rdma(src, j, jj, ext_ref, buf, t):
            return pltpu.make_async_remote_copy(
                src_ref=ext_ref.at[:, pl.ds(HL * j, HL)],
                dst_ref=buf.at[src],
                send_sem=kv_send_sems.at[jj, t],
                recv_sem=kv_recv_sems.at[src, t],
                device_id=(j,),
                device_id_type=pl.DeviceIdType.MESH,
            )

        def loc_copy(src, ext_ref, buf, t):
            return pltpu.make_async_copy(
                ext_ref.at[:, pl.ds(HL * src, HL)],
                buf.at[src],
                loc_sems.at[t],
            )

        for src in range(NSRC):
            @pl.when(me == src)
            def _(src=src):
                jj = 0
                for j in range(N_DEV):
                    if j == src:
                        continue
                    kv_rdma(src, j, jj, kb_ref, kbuf, 0).start()
                    jj += 1
                loc_copy(src, kb_ref, kbuf, 0).start()
                jj = 0
                for j in range(N_DEV):
                    if j == src:
                        continue
                    kv_rdma(src, j, jj, vb_ref, vbuf, 1).start()
                    jj += 1
                loc_copy(src, vb_ref, vbuf, 1).start()

        def wait_kv(t, buf):
            for src in range(NSRC):
                @pl.when(me == src)
                def _(src=src):
                    loc_copy(src, kb_ref if t == 0 else vb_ref, buf, t).wait()

                @pl.when(me != src)
                def _(src=src):
                    pltpu.make_async_remote_copy(
                        src_ref=buf.at[src],
                        dst_ref=buf.at[src],
                        send_sem=kv_send_sems.at[0, t],
                        recv_sem=kv_recv_sems.at[src, t],
                        device_id=(0,),
                        device_id_type=pl.DeviceIdType.MESH,
                    ).wait_recv()

        qi = lax.broadcasted_iota(jnp.int32, (Sq, NSRC * SKVL), 0)
        ki = lax.broadcasted_iota(jnp.int32, (Sq, NSRC * SKVL), 1)
        mask = ki <= qi + WIN

        wait_kv(0, kbuf)
        weights = []
        for b in range(B):
            qb = qm[b * Sq:(b + 1) * Sq, :].reshape(Sq, HL, Dh)
            for h in range(HL):
                q = qb[:, h, :]
                kt = jnp.concatenate(
                    [kbuf[0, b, h], kbuf[1, b, h]], axis=1)
                s = lax.dot_general(
                    q.astype(BF), kt,
                    (((1,), (0,)), ((), ())),
                    preferred_element_type=F32,
                ) * 0.125
                s = jnp.where(mask, s, -1e9)
                m = jnp.max(s, axis=1, keepdims=True)
                e = jnp.exp(s - m)
                w = e / jnp.sum(e, axis=1, keepdims=True)
                weights.append(w.astype(BF))

        wait_kv(1, vbuf)
        ctx_rows = []
        for b in range(B):
            heads = []
            for h in range(HL):
                vt = jnp.concatenate(
                    [vbuf[0, b, h], vbuf[1, b, h]], axis=1)
                ctx = lax.dot_general(
                    weights[b * HL + h], vt,
                    (((1,), (1,)), ((), ())),
                    preferred_element_type=F32)
                heads.append(ctx)
            ctx_rows.append(jnp.concatenate(heads, axis=1))
        ctxm = jnp.concatenate(ctx_rows, axis=0)

        partial = lax.dot(ctxm.astype(BF), wo_ref[...].astype(BF),
                          preferred_element_type=F32)
        pbuf[...] = partial.astype(BF)

        def rs_rdma(o):
            dest = (me + o) % N_DEV
            return pltpu.make_async_remote_copy(
                src_ref=pbuf.at[pl.ds(dest * CH, CH), :],
                dst_ref=rs_buf.at[o],
                send_sem=rs_send_sems.at[o - 1],
                recv_sem=rs_recv_sems.at[o],
                device_id=(dest,),
                device_id_type=pl.DeviceIdType.MESH,
            )

        def ag_rdma(o):
            return pltpu.make_async_remote_copy(
                src_ref=obuf.at[pl.ds(me * CH, CH), :],
                dst_ref=obuf.at[pl.ds(me * CH, CH), :],
                send_sem=ag_send_sems.at[o - 1],
                recv_sem=ag_recv_sems.at[o],
                device_id=((me + o) % N_DEV,),
                device_id_type=pl.DeviceIdType.MESH,
            )

        for o in range(1, N_DEV):
            rs_rdma(o).start()
        rs_buf[0] = pbuf[pl.ds(me * CH, CH), :]
        for o in range(1, N_DEV):
            rs_rdma(o).wait_recv()

        red = jnp.sum(rs_buf[...].astype(F32), axis=0)
        obuf[pl.ds(me * CH, CH), :] = red.astype(BF)

        for o in range(1, N_DEV):
            ag_rdma(o).start()
        for o in range(1, N_DEV):
            ag_rdma(o).wait_recv()

        out_ref[...] = obuf[...].astype(F32).reshape(B, Sq, DM)

        for o in range(1, N_DEV):
            rs_rdma(o).wait_send()
            ag_rdma(o).wait_send()
        for src in range(NSRC):
            @pl.when(me == src)
            def _(src=src):
                jj = 0
                for j in range(N_DEV):
                    if j == src:
                        continue
                    kv_rdma(src, j, jj, kb_ref, kbuf, 0).wait_send()
                    kv_rdma(src, j, jj, vb_ref, vbuf, 1).wait_send()
                    jj += 1

    return pl.pallas_call(
        body,
        out_shape=jax.ShapeDtypeStruct((B, Sq, DM), F32),
        in_specs=[
            pl.BlockSpec(memory_space=pltpu.VMEM),
            pl.BlockSpec(memory_space=pltpu.VMEM),
            pl.BlockSpec(memory_space=pl.ANY),
            pl.BlockSpec(memory_space=pl.ANY),
            pl.BlockSpec(memory_space=pltpu.VMEM),
        ],
        out_specs=pl.BlockSpec(memory_space=pltpu.VMEM),
        scratch_shapes=[
            pltpu.VMEM((NSRC, B, HL, Dh, SKVL), BF),
            pltpu.VMEM((NSRC, B, HL, Dh, SKVL), BF),
            pltpu.VMEM((B * Sq, DM), BF),
            pltpu.VMEM((N_DEV, CH, DM), BF),
            pltpu.VMEM((B * Sq, DM), BF),
            pltpu.SemaphoreType.DMA((N_DEV - 1, 2)),
            pltpu.SemaphoreType.DMA((NSRC, 2)),
            pltpu.SemaphoreType.DMA((2,)),
            pltpu.SemaphoreType.DMA((N_DEV - 1,)),
            pltpu.SemaphoreType.DMA((N_DEV,)),
            pltpu.SemaphoreType.DMA((N_DEV - 1,)),
            pltpu.SemaphoreType.DMA((N_DEV,)),
        ],
        compiler_params=pltpu.CompilerParams(collective_id=0),
    )(x, Wq, Kb, Vb, Wo)
